# baseline (speedup 1.0000x reference)
import functools

import jax
import jax.numpy as jnp
from jax.experimental import pallas as pl
from jax.experimental.pallas import tpu as pltpu

_E = 2
_P = 8


def _down3d_block_kernel(x_ref, wdw1_ref, s1_ref, b1_ref, m1_ref,
                         wdw2_ref, s2_ref, b2_ref, m2_ref, mres_ref,
                         o_ref, xp_ref, y1_ref, *, D, H, W, L, Lo):
    e = _E
    R = D * H * W
    bf16 = jnp.bfloat16

    xp_ref[...] = jnp.zeros_like(xp_ref)
    xp_ref[e:e + D, e:e + H, e:e + W, :] = x_ref[...]

    def dw27(src_ref, w_ref):
        acc = None
        for kd in range(3):
            for kh in range(3):
                for kw in range(3):
                    tap = src_ref[kd * e:kd * e + D, kh * e:kh * e + H,
                                  kw * e:kw * e + W, :]
                    t = tap * w_ref[kd, kh * 3 + kw]
                    acc = t if acc is None else acc + t
        return acc

    z1 = jnp.clip(dw27(xp_ref, wdw1_ref) * s1_ref[0] + b1_ref[0], 0.0, 6.0)
    y1 = jnp.dot(z1.reshape(R, L).astype(bf16), m1_ref[...],
                 preferred_element_type=jnp.float32)

    y1_ref[...] = jnp.zeros_like(y1_ref)
    y1_ref[e:e + D, e:e + H, e:e + W, :] = y1.reshape(D, H, W, L)

    z2 = jnp.clip(dw27(y1_ref, wdw2_ref) * s2_ref[0] + b2_ref[0], 0.0, 6.0)
    r2 = jnp.dot(z2.reshape(R, L).astype(bf16), m2_ref[...],
                 preferred_element_type=jnp.float32)

    rr = jnp.dot(x_ref[...].reshape(R, L).astype(bf16), mres_ref[...],
                 preferred_element_type=jnp.float32)

    o_ref[...] = (r2 + rr).reshape(D, H, W, Lo)


def _down3d(xt, wdw1, s1, b1, m1, wdw2, s2, b2, m2, mres,
            *, D, H, W, L, Lo):
    G = xt.shape[0]
    e = _E
    Dh, Hh, Wh = D + 2 * e, H + 2 * e, W + 2 * e

    kern = functools.partial(_down3d_block_kernel, D=D, H=H, W=W, L=L, Lo=Lo)
    zero2 = lambda i: (0, 0)
    zero3 = lambda i: (0, 0, 0)

    return pl.pallas_call(
        kern,
        out_shape=jax.ShapeDtypeStruct((G, D, H, W, Lo), jnp.float32),
        grid=(G,),
        in_specs=[
            pl.BlockSpec((None, D, H, W, L), lambda i: (i, 0, 0, 0, 0)),
            pl.BlockSpec((3, 9, L), zero3),
            pl.BlockSpec((1, L), zero2),
            pl.BlockSpec((1, L), zero2),
            pl.BlockSpec((L, L), zero2),
            pl.BlockSpec((3, 9, L), zero3),
            pl.BlockSpec((1, L), zero2),
            pl.BlockSpec((1, L), zero2),
            pl.BlockSpec((L, Lo), zero2),
            pl.BlockSpec((L, Lo), zero2),
        ],
        out_specs=pl.BlockSpec((None, D, H, W, Lo), lambda i: (i, 0, 0, 0, 0)),
        scratch_shapes=[
            pltpu.VMEM((Dh, Hh, Wh, L), jnp.float32),
            pltpu.VMEM((Dh, Hh, Wh, L), jnp.float32),
        ],
        compiler_params=pltpu.CompilerParams(
            dimension_semantics=("parallel",),
            vmem_limit_bytes=56 << 20),
    )(xt, wdw1, s1, b1, m1, wdw2, s2, b2, m2, mres)


def kernel(x, w_dw1, w_pw1, scale1, bias1, w_dw2, w_pw2, scale2, bias2,
           w_res):
    N, Cin, D, H, W = x.shape
    Cout = w_res.shape[1]
    f32, bf16 = jnp.float32, jnp.bfloat16
    P = _P
    L, Lo = P * Cin, P * Cout

    xt = (x.astype(f32)
          .reshape(N // P, P, Cin, D, H, W)
          .transpose(0, 3, 4, 5, 1, 2)
          .reshape(N // P, D, H, W, L))

    wdw1 = jnp.tile(w_dw1.astype(f32), (1, P)).reshape(3, 9, L)
    wdw2 = jnp.tile(w_dw2.astype(f32), (1, P)).reshape(3, 9, L)
    s1 = jnp.tile(scale1.reshape(1, Cin).astype(f32), (1, P))
    b1 = jnp.tile(bias1.reshape(1, Cin).astype(f32), (1, P))
    s2 = jnp.tile(scale2.reshape(1, Cin).astype(f32), (1, P))
    b2 = jnp.tile(bias2.reshape(1, Cin).astype(f32), (1, P))

    eye = jnp.eye(P, dtype=f32)
    m1 = jnp.kron(eye, w_pw1.astype(f32)).astype(bf16)
    m2 = jnp.kron(eye, w_pw2.astype(f32)).astype(bf16)
    mres = jnp.kron(eye, w_res.astype(f32)).astype(bf16)

    out = _down3d(xt, wdw1, s1, b1, m1, wdw2, s2, b2, m2, mres,
                  D=D, H=H, W=W, L=L, Lo=Lo)
    out = (out.reshape(N // P, D, H, W, P, Cout)
           .transpose(0, 4, 5, 1, 2, 3)
           .reshape(N, Cout, D, H, W))
    return out


# speedup vs baseline: 1.3512x; 1.3512x over previous
import functools

import jax
import jax.numpy as jnp
from jax.experimental import pallas as pl
from jax.experimental.pallas import tpu as pltpu

_E = 2
_P = 8


def _down3d_block_kernel(x_ref, wdw1_ref, s1_ref, b1_ref, m1_ref,
                         wdw2_ref, s2_ref, b2_ref, m2t_ref, mrest_ref,
                         i128_ref, o_ref, xp_ref, y1_ref, xb_ref,
                         *, D, H, W, L, Lo):
    e = _E
    R = D * H * W
    HW = H * W
    bf16 = jnp.bfloat16
    f32 = jnp.float32

    xp_ref[...] = jnp.zeros_like(xp_ref)
    for d in range(D):
        xd = x_ref[d].astype(bf16)
        xb_ref[d] = xd
        td = jnp.einsum("km,kn->mn", xd, i128_ref[...],
                        preferred_element_type=f32)
        xp_ref[e + d, e:e + H, e:e + W, :] = td.reshape(H, W, L)

    def dw27(src_ref, w_ref):
        acc = None
        for kd in range(3):
            for kh in range(3):
                for kw in range(3):
                    tap = src_ref[kd * e:kd * e + D, kh * e:kh * e + H,
                                  kw * e:kw * e + W, :]
                    t = tap * w_ref[kd, kh * 3 + kw]
                    acc = t if acc is None else acc + t
        return acc

    z1 = jnp.clip(dw27(xp_ref, wdw1_ref) * s1_ref[0] + b1_ref[0], 0.0, 6.0)
    y1 = jnp.dot(z1.reshape(R, L).astype(bf16), m1_ref[...],
                 preferred_element_type=f32)

    y1_ref[...] = jnp.zeros_like(y1_ref)
    y1_ref[e:e + D, e:e + H, e:e + W, :] = y1.reshape(D, H, W, L)

    z2 = jnp.clip(dw27(y1_ref, wdw2_ref) * s2_ref[0] + b2_ref[0], 0.0, 6.0)
    z2b = z2.reshape(R, L).astype(bf16)
    for d in range(D):
        z2d = z2b[d * HW:(d + 1) * HW]
        z2t = jnp.einsum("mk,nk->mn", i128_ref[...], z2d,
                         preferred_element_type=f32)
        r2t = jnp.dot(m2t_ref[...], z2t.astype(bf16),
                      preferred_element_type=f32)
        rrt = jnp.dot(mrest_ref[...], xb_ref[d],
                      preferred_element_type=f32)
        o_ref[d] = r2t + rrt


def _down3d(x4, wdw1, s1, b1, m1, wdw2, s2, b2, m2t, mrest, i128,
            *, D, H, W, L, Lo):
    G = x4.shape[0]
    e = _E
    HW = H * W
    Dh, Hh, Wh = D + 2 * e, H + 2 * e, W + 2 * e

    kern = functools.partial(_down3d_block_kernel, D=D, H=H, W=W, L=L, Lo=Lo)
    zero2 = lambda i: (0, 0)
    zero3 = lambda i: (0, 0, 0)

    return pl.pallas_call(
        kern,
        out_shape=jax.ShapeDtypeStruct((G, D, Lo, HW), jnp.float32),
        grid=(G,),
        in_specs=[
            pl.BlockSpec((None, D, L, HW), lambda i: (i, 0, 0, 0)),
            pl.BlockSpec((3, 9, L), zero3),
            pl.BlockSpec((1, L), zero2),
            pl.BlockSpec((1, L), zero2),
            pl.BlockSpec((L, L), zero2),
            pl.BlockSpec((3, 9, L), zero3),
            pl.BlockSpec((1, L), zero2),
            pl.BlockSpec((1, L), zero2),
            pl.BlockSpec((Lo, L), zero2),
            pl.BlockSpec((Lo, L), zero2),
            pl.BlockSpec((L, L), zero2),
        ],
        out_specs=pl.BlockSpec((None, D, Lo, HW), lambda i: (i, 0, 0, 0)),
        scratch_shapes=[
            pltpu.VMEM((Dh, Hh, Wh, L), jnp.float32),
            pltpu.VMEM((Dh, Hh, Wh, L), jnp.float32),
            pltpu.VMEM((D, L, HW), jnp.bfloat16),
        ],
        compiler_params=pltpu.CompilerParams(
            dimension_semantics=("parallel",),
            vmem_limit_bytes=56 << 20),
    )(x4, wdw1, s1, b1, m1, wdw2, s2, b2, m2t, mrest, i128)


def kernel(x, w_dw1, w_pw1, scale1, bias1, w_dw2, w_pw2, scale2, bias2,
           w_res):
    N, Cin, D, H, W = x.shape
    Cout = w_res.shape[1]
    f32, bf16 = jnp.float32, jnp.bfloat16
    P = _P
    L, Lo = P * Cin, P * Cout
    G = N // P

    xr = x.astype(f32).reshape(G, L, D, H * W)
    x4 = jnp.transpose(xr, (0, 2, 1, 3))

    wdw1 = jnp.tile(w_dw1.astype(f32), (1, P)).reshape(3, 9, L)
    wdw2 = jnp.tile(w_dw2.astype(f32), (1, P)).reshape(3, 9, L)
    s1 = jnp.tile(scale1.reshape(1, Cin).astype(f32), (1, P))
    b1 = jnp.tile(bias1.reshape(1, Cin).astype(f32), (1, P))
    s2 = jnp.tile(scale2.reshape(1, Cin).astype(f32), (1, P))
    b2 = jnp.tile(bias2.reshape(1, Cin).astype(f32), (1, P))

    eye = jnp.eye(P, dtype=f32)
    m1 = jnp.kron(eye, w_pw1.astype(f32)).astype(bf16)
    m2t = jnp.kron(eye, w_pw2.astype(f32)).T.astype(bf16)
    mrest = jnp.kron(eye, w_res.astype(f32)).T.astype(bf16)
    i128 = jnp.eye(L, dtype=bf16)

    out = _down3d(x4, wdw1, s1, b1, m1, wdw2, s2, b2, m2t, mrest, i128,
                  D=D, H=H, W=W, L=L, Lo=Lo)
    out = jnp.transpose(out, (0, 2, 1, 3))
    return out.reshape(N, Cout, D, H, W)
